# Initial kernel scaffold; baseline (speedup 1.0000x reference)
import jax
import jax.numpy as jnp
from jax.experimental import pallas as pl


def kernel(x, pos_table, ln_gamma, ln_beta, batch_size):
    raise NotImplementedError("write your pallas kernel here")


# speedup vs baseline: 3.5534x; 3.5534x over previous
import jax
import jax.numpy as jnp
from jax.experimental import pallas as pl

_EPS = 1e-5
_BLOCK_S = 512


def _embed_ln_body(x_ref, pos_ref, g_ref, b_ref, o_ref):
    v = x_ref[0] + pos_ref[...]
    mean = jnp.mean(v, axis=-1, keepdims=True)
    c = v - mean
    var = jnp.mean(c * c, axis=-1, keepdims=True)
    o_ref[0] = c * jax.lax.rsqrt(var + _EPS) * g_ref[...] + b_ref[...]


def kernel(x, pos_table, ln_gamma, ln_beta, batch_size):
    del batch_size
    B, S, D = x.shape
    g2 = ln_gamma.reshape(1, D)
    b2 = ln_beta.reshape(1, D)
    return pl.pallas_call(
        _embed_ln_body,
        grid=(S // _BLOCK_S, B),
        in_specs=[
            pl.BlockSpec((1, _BLOCK_S, D), lambda i, j: (j, i, 0)),
            pl.BlockSpec((_BLOCK_S, D), lambda i, j: (i, 0)),
            pl.BlockSpec((1, D), lambda i, j: (0, 0)),
            pl.BlockSpec((1, D), lambda i, j: (0, 0)),
        ],
        out_specs=pl.BlockSpec((1, _BLOCK_S, D), lambda i, j: (j, i, 0)),
        out_shape=jax.ShapeDtypeStruct((B, S, D), x.dtype),
    )(x, pos_table, g2, b2)


# speedup vs baseline: 4.0712x; 1.1457x over previous
import jax
import jax.numpy as jnp
from jax.experimental import pallas as pl

_EPS = 1e-5
_BLOCK_S = 1024


def _embed_ln_body(x_ref, pos_ref, g_ref, b_ref, o_ref):
    v = x_ref[0] + pos_ref[...]
    mean = jnp.mean(v, axis=-1, keepdims=True)
    c = v - mean
    var = jnp.mean(c * c, axis=-1, keepdims=True)
    o_ref[0] = c * jax.lax.rsqrt(var + _EPS) * g_ref[...] + b_ref[...]


def kernel(x, pos_table, ln_gamma, ln_beta, batch_size):
    del batch_size
    B, S, D = x.shape
    g2 = ln_gamma.reshape(1, D)
    b2 = ln_beta.reshape(1, D)
    return pl.pallas_call(
        _embed_ln_body,
        grid=(S // _BLOCK_S, B),
        in_specs=[
            pl.BlockSpec((1, _BLOCK_S, D), lambda i, j: (j, i, 0)),
            pl.BlockSpec((_BLOCK_S, D), lambda i, j: (i, 0)),
            pl.BlockSpec((1, D), lambda i, j: (0, 0)),
            pl.BlockSpec((1, D), lambda i, j: (0, 0)),
        ],
        out_specs=pl.BlockSpec((1, _BLOCK_S, D), lambda i, j: (j, i, 0)),
        out_shape=jax.ShapeDtypeStruct((B, S, D), x.dtype),
    )(x, pos_table, g2, b2)


# speedup vs baseline: 4.2376x; 1.0409x over previous
import jax
import jax.numpy as jnp
from jax.experimental import pallas as pl

_EPS = 1e-5
_BLOCK_S = 2048


def _embed_ln_body(x_ref, pos_ref, g_ref, b_ref, o_ref):
    v = x_ref[0] + pos_ref[...]
    mean = jnp.mean(v, axis=-1, keepdims=True)
    c = v - mean
    var = jnp.mean(c * c, axis=-1, keepdims=True)
    o_ref[0] = c * jax.lax.rsqrt(var + _EPS) * g_ref[...] + b_ref[...]


def kernel(x, pos_table, ln_gamma, ln_beta, batch_size):
    del batch_size
    B, S, D = x.shape
    g2 = ln_gamma.reshape(1, D)
    b2 = ln_beta.reshape(1, D)
    return pl.pallas_call(
        _embed_ln_body,
        grid=(S // _BLOCK_S, B),
        in_specs=[
            pl.BlockSpec((1, _BLOCK_S, D), lambda i, j: (j, i, 0)),
            pl.BlockSpec((_BLOCK_S, D), lambda i, j: (i, 0)),
            pl.BlockSpec((1, D), lambda i, j: (0, 0)),
            pl.BlockSpec((1, D), lambda i, j: (0, 0)),
        ],
        out_specs=pl.BlockSpec((1, _BLOCK_S, D), lambda i, j: (j, i, 0)),
        out_shape=jax.ShapeDtypeStruct((B, S, D), x.dtype),
    )(x, pos_table, g2, b2)


# speedup vs baseline: 4.5231x; 1.0674x over previous
import jax
import jax.numpy as jnp
from jax.experimental import pallas as pl

_EPS = 1e-5
_BLOCK_S = 512


def _embed_ln_body(x_ref, pos_ref, g_ref, b_ref, o_ref):
    v = x_ref[...] + pos_ref[...][None, :, :]
    mean = jnp.mean(v, axis=-1, keepdims=True)
    c = v - mean
    var = jnp.mean(c * c, axis=-1, keepdims=True)
    o_ref[...] = c * jax.lax.rsqrt(var + _EPS) * g_ref[...] + b_ref[...]


def kernel(x, pos_table, ln_gamma, ln_beta, batch_size):
    del batch_size
    B, S, D = x.shape
    g2 = ln_gamma.reshape(1, 1, D)
    b2 = ln_beta.reshape(1, 1, D)
    return pl.pallas_call(
        _embed_ln_body,
        grid=(S // _BLOCK_S,),
        in_specs=[
            pl.BlockSpec((B, _BLOCK_S, D), lambda i: (0, i, 0)),
            pl.BlockSpec((_BLOCK_S, D), lambda i: (i, 0)),
            pl.BlockSpec((1, 1, D), lambda i: (0, 0, 0)),
            pl.BlockSpec((1, 1, D), lambda i: (0, 0, 0)),
        ],
        out_specs=pl.BlockSpec((B, _BLOCK_S, D), lambda i: (0, i, 0)),
        out_shape=jax.ShapeDtypeStruct((B, S, D), x.dtype),
    )(x, pos_table, g2, b2)
